# Initial kernel scaffold; baseline (speedup 1.0000x reference)
import jax
import jax.numpy as jnp
from jax.experimental import pallas as pl


def kernel(x, params):
    raise NotImplementedError("write your pallas kernel here")


# speedup vs baseline: 574.3889x; 574.3889x over previous
import functools

import jax
import jax.numpy as jnp
from jax import lax
from jax.experimental import pallas as pl
from jax.experimental.pallas import tpu as pltpu
from jax.experimental.pallas import tpu_sc as plsc

RANGE_MIN = -5.0
RANGE_MAX = 5.0
MIN_BIN_SIZE = 0.0001
MIN_SLOPE = 0.0001

LN2 = 0.6931471805599453
SQRT2 = 1.4142135623730951

N = 8388608
NC, NS, L = 2, 16, 16
NW = NC * NS
PER_W = N // NW
CHUNK = 32768
N_CHUNKS = PER_W // CHUNK
VREGS = CHUNK // L
TPAD = 144


def _log_approx(t):
    bits = lax.bitcast_convert_type(t, jnp.int32)
    e_i = (bits >> 23) - 127
    m = lax.bitcast_convert_type((bits & 0x007FFFFF) | 0x3F800000, jnp.float32)
    big = m >= SQRT2
    m = jnp.where(big, m * 0.5, m)
    e_f = e_i.astype(jnp.float32) + jnp.where(big, 1.0, 0.0)
    z = (m - 1.0) / (m + 1.0)
    z2 = z * z
    p = z * (2.0 + z2 * (2.0 / 3.0 + z2 * (2.0 / 5.0 + z2 * (2.0 / 7.0))))
    return e_f * LN2 + p


def _sc_body(x_hbm, xpos_hbm, ypos_hbm, d_hbm, invw_hbm, h_hbm,
             y_hbm, ld_hbm,
             xpos_v, ypos_v, d_v, invw_v, h_v, x_v, y_v, ld_v):
    wid = lax.axis_index("s") * NC + lax.axis_index("c")
    base = wid * PER_W

    pltpu.sync_copy(xpos_hbm, xpos_v)
    pltpu.sync_copy(ypos_hbm, ypos_v)
    pltpu.sync_copy(d_hbm, d_v)
    pltpu.sync_copy(invw_hbm, invw_v)
    pltpu.sync_copy(h_hbm, h_v)

    def vreg_body(i, carry):
        off = i * L
        xv = x_v[pl.ds(off, L)]
        b = jnp.zeros((L,), jnp.int32)
        for step in (64, 32, 16, 8, 4, 2, 1):
            cand = b + step
            knot = plsc.load_gather(xpos_v, [cand])
            b = jnp.where(knot <= xv, cand, b)
        x_k = plsc.load_gather(xpos_v, [b])
        y_k = plsc.load_gather(ypos_v, [b])
        iw = plsc.load_gather(invw_v, [b])
        hh = plsc.load_gather(h_v, [b])
        d_k = plsc.load_gather(d_v, [b])
        d_k1 = plsc.load_gather(d_v, [b + 1])
        s_ = hh * iw
        xi = jnp.clip((xv - x_k) * iw, 0.0, 1.0)
        om = 1.0 - xi
        xiom = xi * om
        num = xi * (s_ * xi + d_k * om)
        den = s_ + (d_k1 + d_k - 2.0 * s_) * xiom
        rden = 1.0 / den
        y_spline = y_k + hh * num * rden
        numd = s_ * s_ * (d_k1 * xi * xi + 2.0 * s_ * xiom + d_k * om * om)
        deriv = numd * rden * rden
        below = xv < RANGE_MIN
        above = xv > RANGE_MAX
        yv = jnp.where(below, (xv - RANGE_MIN) * d_k + RANGE_MIN,
                       jnp.where(above, (xv - RANGE_MAX) * d_k1 + RANGE_MAX,
                                 y_spline))
        tv = jnp.where(below, d_k, jnp.where(above, d_k1, deriv))
        y_v[pl.ds(off, L)] = yv
        ld_v[pl.ds(off, L)] = _log_approx(tv)
        return carry

    for c in range(N_CHUNKS):
        lo = base + c * CHUNK
        pltpu.sync_copy(x_hbm.at[pl.ds(lo, CHUNK)], x_v)
        lax.fori_loop(0, VREGS, vreg_body, 0)
        pltpu.sync_copy(y_v, y_hbm.at[pl.ds(lo, CHUNK)])
        pltpu.sync_copy(ld_v, ld_hbm.at[pl.ds(lo, CHUNK)])


@jax.jit
def kernel(x, params):
    K = (params.shape[-1] - 1) // 3
    total_size = RANGE_MAX - RANGE_MIN
    widths = jax.nn.softmax(params[:K]) * (total_size - K * MIN_BIN_SIZE) + MIN_BIN_SIZE
    heights = jax.nn.softmax(params[K:2 * K]) * (total_size - K * MIN_BIN_SIZE) + MIN_BIN_SIZE
    slopes_offset = jnp.log(jnp.exp(1.0 - MIN_SLOPE) - 1.0)
    slopes = jax.nn.softplus(params[2 * K:] + slopes_offset) + MIN_SLOPE
    x_pos = jnp.concatenate([jnp.array([0.0]), jnp.cumsum(widths)]) + RANGE_MIN
    y_pos = jnp.concatenate([jnp.array([0.0]), jnp.cumsum(heights)]) + RANGE_MIN

    def padto(a):
        return jnp.pad(a, (0, TPAD - a.shape[0]), constant_values=1.0).astype(jnp.float32)

    xpos_p = padto(x_pos)
    ypos_p = padto(y_pos)
    d_p = padto(slopes)
    invw_p = padto(1.0 / (x_pos[1:] - x_pos[:-1]))
    h_p = padto(y_pos[1:] - y_pos[:-1])

    mesh = plsc.VectorSubcoreMesh(core_axis_name="c", subcore_axis_name="s")
    f32 = jnp.float32
    run = pl.kernel(
        _sc_body,
        mesh=mesh,
        compiler_params=pltpu.CompilerParams(needs_layout_passes=False),
        out_type=(jax.ShapeDtypeStruct((N,), f32),
                  jax.ShapeDtypeStruct((N,), f32)),
        scratch_types=[
            pltpu.VMEM((TPAD,), f32),
            pltpu.VMEM((TPAD,), f32),
            pltpu.VMEM((TPAD,), f32),
            pltpu.VMEM((TPAD,), f32),
            pltpu.VMEM((TPAD,), f32),
            pltpu.VMEM((CHUNK,), f32),
            pltpu.VMEM((CHUNK,), f32),
            pltpu.VMEM((CHUNK,), f32),
        ],
    )
    return run(x, xpos_p, ypos_p, d_p, invw_p, h_p)


# speedup vs baseline: 1409.3280x; 2.4536x over previous
import functools

import jax
import jax.numpy as jnp
from jax import lax
from jax.experimental import pallas as pl
from jax.experimental.pallas import tpu as pltpu
from jax.experimental.pallas import tpu_sc as plsc

RANGE_MIN = -5.0
RANGE_MAX = 5.0
MIN_BIN_SIZE = 0.0001
MIN_SLOPE = 0.0001

LN2 = 0.6931471805599453
SQRT2 = 1.4142135623730951

N = 8388608
NC, NS, L = 2, 16, 16
NW = NC * NS
PER_W = N // NW
CHUNK = 32768
N_CHUNKS = PER_W // CHUNK
VREGS = CHUNK // L
TPAD = 144


def _log_approx(t):
    bits = lax.bitcast_convert_type(t, jnp.int32)
    e_i = (bits >> 23) - 127
    m = lax.bitcast_convert_type((bits & 0x007FFFFF) | 0x3F800000, jnp.float32)
    big = m >= SQRT2
    m = jnp.where(big, m * 0.5, m)
    e_f = e_i.astype(jnp.float32) + jnp.where(big, 1.0, 0.0)
    z = (m - 1.0) / (m + 1.0)
    z2 = z * z
    p = z * (2.0 + z2 * (2.0 / 3.0 + z2 * (2.0 / 5.0 + z2 * (2.0 / 7.0))))
    return e_f * LN2 + p


def _sc_body(x_hbm, xpos_hbm, ypos_hbm, d_hbm, invw_hbm, h_hbm,
             y_hbm, ld_hbm,
             xpos_v, ypos_v, d_v, invw_v, h_v, x_v, y_v, ld_v):
    wid = lax.axis_index("s") * NC + lax.axis_index("c")
    base = wid * PER_W

    pltpu.sync_copy(xpos_hbm, xpos_v)
    pltpu.sync_copy(ypos_hbm, ypos_v)
    pltpu.sync_copy(d_hbm, d_v)
    pltpu.sync_copy(invw_hbm, invw_v)
    pltpu.sync_copy(h_hbm, h_v)

    def vreg_body(off):
        xv = x_v[pl.ds(off, L)]
        b = jnp.zeros((L,), jnp.int32)
        for step in (64, 32, 16, 8, 4, 2, 1):
            cand = b + step
            knot = plsc.load_gather(xpos_v, [cand])
            b = jnp.where(knot <= xv, cand, b)
        x_k = plsc.load_gather(xpos_v, [b])
        y_k = plsc.load_gather(ypos_v, [b])
        iw = plsc.load_gather(invw_v, [b])
        hh = plsc.load_gather(h_v, [b])
        d_k = plsc.load_gather(d_v, [b])
        d_k1 = plsc.load_gather(d_v, [b + 1])
        s_ = hh * iw
        xi = jnp.clip((xv - x_k) * iw, 0.0, 1.0)
        om = 1.0 - xi
        xiom = xi * om
        num = xi * (s_ * xi + d_k * om)
        den = s_ + (d_k1 + d_k - 2.0 * s_) * xiom
        rden = 1.0 / den
        y_spline = y_k + hh * num * rden
        numd = s_ * s_ * (d_k1 * xi * xi + 2.0 * s_ * xiom + d_k * om * om)
        deriv = numd * rden * rden
        below = xv < RANGE_MIN
        above = xv > RANGE_MAX
        yv = jnp.where(below, (xv - RANGE_MIN) * d_k + RANGE_MIN,
                       jnp.where(above, (xv - RANGE_MAX) * d_k1 + RANGE_MAX,
                                 y_spline))
        tv = jnp.where(below, d_k, jnp.where(above, d_k1, deriv))
        y_v[pl.ds(off, L)] = yv
        ld_v[pl.ds(off, L)] = _log_approx(tv)

    def chunk_body(c, carry):
        lo = base + c * CHUNK
        pltpu.sync_copy(x_hbm.at[pl.ds(lo, CHUNK)], x_v)
        plsc.parallel_loop(0, CHUNK, L, unroll=8)(vreg_body)
        pltpu.sync_copy(y_v, y_hbm.at[pl.ds(lo, CHUNK)])
        pltpu.sync_copy(ld_v, ld_hbm.at[pl.ds(lo, CHUNK)])
        return carry

    lax.fori_loop(0, N_CHUNKS, chunk_body, 0)


@jax.jit
def kernel(x, params):
    K = (params.shape[-1] - 1) // 3
    total_size = RANGE_MAX - RANGE_MIN
    widths = jax.nn.softmax(params[:K]) * (total_size - K * MIN_BIN_SIZE) + MIN_BIN_SIZE
    heights = jax.nn.softmax(params[K:2 * K]) * (total_size - K * MIN_BIN_SIZE) + MIN_BIN_SIZE
    slopes_offset = jnp.log(jnp.exp(1.0 - MIN_SLOPE) - 1.0)
    slopes = jax.nn.softplus(params[2 * K:] + slopes_offset) + MIN_SLOPE
    x_pos = jnp.concatenate([jnp.array([0.0]), jnp.cumsum(widths)]) + RANGE_MIN
    y_pos = jnp.concatenate([jnp.array([0.0]), jnp.cumsum(heights)]) + RANGE_MIN

    def padto(a):
        return jnp.pad(a, (0, TPAD - a.shape[0]), constant_values=1.0).astype(jnp.float32)

    xpos_p = padto(x_pos)
    ypos_p = padto(y_pos)
    d_p = padto(slopes)
    invw_p = padto(1.0 / (x_pos[1:] - x_pos[:-1]))
    h_p = padto(y_pos[1:] - y_pos[:-1])

    mesh = plsc.VectorSubcoreMesh(core_axis_name="c", subcore_axis_name="s")
    f32 = jnp.float32
    run = pl.kernel(
        _sc_body,
        mesh=mesh,
        compiler_params=pltpu.CompilerParams(needs_layout_passes=False),
        out_type=(jax.ShapeDtypeStruct((N,), f32),
                  jax.ShapeDtypeStruct((N,), f32)),
        scratch_types=[
            pltpu.VMEM((TPAD,), f32),
            pltpu.VMEM((TPAD,), f32),
            pltpu.VMEM((TPAD,), f32),
            pltpu.VMEM((TPAD,), f32),
            pltpu.VMEM((TPAD,), f32),
            pltpu.VMEM((CHUNK,), f32),
            pltpu.VMEM((CHUNK,), f32),
            pltpu.VMEM((CHUNK,), f32),
        ],
    )
    return run(x, xpos_p, ypos_p, d_p, invw_p, h_p)


# speedup vs baseline: 1457.8919x; 1.0345x over previous
import functools

import jax
import jax.numpy as jnp
from jax import lax
from jax.experimental import pallas as pl
from jax.experimental.pallas import tpu as pltpu
from jax.experimental.pallas import tpu_sc as plsc

RANGE_MIN = -5.0
RANGE_MAX = 5.0
MIN_BIN_SIZE = 0.0001
MIN_SLOPE = 0.0001

LN2 = 0.6931471805599453
SQRT2 = 1.4142135623730951

N = 8388608
NC, NS, L = 2, 16, 16
NW = NC * NS
PER_W = N // NW
CHUNK = 16384
N_CHUNKS = PER_W // CHUNK
N_STEPS = N_CHUNKS // 2
VREGS = CHUNK // L
TPAD = 144


def _log_approx(t):
    bits = lax.bitcast_convert_type(t, jnp.int32)
    e_i = (bits >> 23) - 127
    m = lax.bitcast_convert_type((bits & 0x007FFFFF) | 0x3F800000, jnp.float32)
    big = m >= SQRT2
    m = jnp.where(big, m * 0.5, m)
    e_f = e_i.astype(jnp.float32) + jnp.where(big, 1.0, 0.0)
    z = (m - 1.0) / (m + 1.0)
    z2 = z * z
    p = z * (2.0 + z2 * (2.0 / 3.0 + z2 * (2.0 / 5.0 + z2 * (2.0 / 7.0))))
    return e_f * LN2 + p


def _sc_body(x_hbm, xpos_hbm, ypos_hbm, d_hbm, invw_hbm, h_hbm,
             y_hbm, ld_hbm,
             xpos_v, ypos_v, d_v, invw_v, h_v,
             x0, x1, y0, y1, l0, l1,
             sem_in0, sem_in1, sem_oy0, sem_oy1, sem_ol0, sem_ol1):
    wid = lax.axis_index("s") * NC + lax.axis_index("c")
    base = wid * PER_W

    pltpu.sync_copy(xpos_hbm, xpos_v)
    pltpu.sync_copy(ypos_hbm, ypos_v)
    pltpu.sync_copy(d_hbm, d_v)
    pltpu.sync_copy(invw_hbm, invw_v)
    pltpu.sync_copy(h_hbm, h_v)

    def make_vreg_body(x_v, y_v, ld_v):
      def vreg_body(off):
        xv = x_v[pl.ds(off, L)]
        b = jnp.zeros((L,), jnp.int32)
        for step in (64, 32, 16, 8, 4, 2, 1):
            cand = b + step
            knot = plsc.load_gather(xpos_v, [cand])
            b = jnp.where(knot <= xv, cand, b)
        x_k = plsc.load_gather(xpos_v, [b])
        y_k = plsc.load_gather(ypos_v, [b])
        iw = plsc.load_gather(invw_v, [b])
        hh = plsc.load_gather(h_v, [b])
        d_k = plsc.load_gather(d_v, [b])
        d_k1 = plsc.load_gather(d_v, [b + 1])
        s_ = hh * iw
        xi = jnp.clip((xv - x_k) * iw, 0.0, 1.0)
        om = 1.0 - xi
        xiom = xi * om
        num = xi * (s_ * xi + d_k * om)
        den = s_ + (d_k1 + d_k - 2.0 * s_) * xiom
        rden = 1.0 / den
        y_spline = y_k + hh * num * rden
        numd = s_ * s_ * (d_k1 * xi * xi + 2.0 * s_ * xiom + d_k * om * om)
        deriv = numd * rden * rden
        below = xv < RANGE_MIN
        above = xv > RANGE_MAX
        yv = jnp.where(below, (xv - RANGE_MIN) * d_k + RANGE_MIN,
                       jnp.where(above, (xv - RANGE_MAX) * d_k1 + RANGE_MAX,
                                 y_spline))
        tv = jnp.where(below, d_k, jnp.where(above, d_k1, deriv))
        y_v[pl.ds(off, L)] = yv
        ld_v[pl.ds(off, L)] = _log_approx(tv)
      return vreg_body

    def half(i, g, x_v, y_v, ld_v, sem_in, sem_oy, sem_ol):
        lo = base + g * CHUNK
        out_y = pltpu.make_async_copy(y_v, y_hbm.at[pl.ds(lo, CHUNK)], sem_oy)
        out_l = pltpu.make_async_copy(ld_v, ld_hbm.at[pl.ds(lo, CHUNK)], sem_ol)

        @pl.when(i > 0)
        def _():
            out_y.wait()
            out_l.wait()

        pltpu.make_async_copy(x_hbm.at[pl.ds(lo, CHUNK)], x_v, sem_in).wait()
        plsc.parallel_loop(0, CHUNK, L, unroll=8)(make_vreg_body(x_v, y_v, ld_v))
        out_y.start()
        out_l.start()

        @pl.when(i < N_STEPS - 1)
        def _():
            nxt = lo + 2 * CHUNK
            pltpu.make_async_copy(x_hbm.at[pl.ds(nxt, CHUNK)], x_v, sem_in).start()

    pltpu.make_async_copy(x_hbm.at[pl.ds(base, CHUNK)], x0, sem_in0).start()
    pltpu.make_async_copy(x_hbm.at[pl.ds(base + CHUNK, CHUNK)], x1, sem_in1).start()

    def step(i, carry):
        half(i, 2 * i, x0, y0, l0, sem_in0, sem_oy0, sem_ol0)
        half(i, 2 * i + 1, x1, y1, l1, sem_in1, sem_oy1, sem_ol1)
        return carry

    lax.fori_loop(0, N_STEPS, step, 0)

    tail = base + (N_CHUNKS - 2) * CHUNK
    pltpu.make_async_copy(y0, y_hbm.at[pl.ds(tail, CHUNK)], sem_oy0).wait()
    pltpu.make_async_copy(l0, ld_hbm.at[pl.ds(tail, CHUNK)], sem_ol0).wait()
    pltpu.make_async_copy(y1, y_hbm.at[pl.ds(tail + CHUNK, CHUNK)], sem_oy1).wait()
    pltpu.make_async_copy(l1, ld_hbm.at[pl.ds(tail + CHUNK, CHUNK)], sem_ol1).wait()


@jax.jit
def kernel(x, params):
    K = (params.shape[-1] - 1) // 3
    total_size = RANGE_MAX - RANGE_MIN
    widths = jax.nn.softmax(params[:K]) * (total_size - K * MIN_BIN_SIZE) + MIN_BIN_SIZE
    heights = jax.nn.softmax(params[K:2 * K]) * (total_size - K * MIN_BIN_SIZE) + MIN_BIN_SIZE
    slopes_offset = jnp.log(jnp.exp(1.0 - MIN_SLOPE) - 1.0)
    slopes = jax.nn.softplus(params[2 * K:] + slopes_offset) + MIN_SLOPE
    x_pos = jnp.concatenate([jnp.array([0.0]), jnp.cumsum(widths)]) + RANGE_MIN
    y_pos = jnp.concatenate([jnp.array([0.0]), jnp.cumsum(heights)]) + RANGE_MIN

    def padto(a):
        return jnp.pad(a, (0, TPAD - a.shape[0]), constant_values=1.0).astype(jnp.float32)

    xpos_p = padto(x_pos)
    ypos_p = padto(y_pos)
    d_p = padto(slopes)
    invw_p = padto(1.0 / (x_pos[1:] - x_pos[:-1]))
    h_p = padto(y_pos[1:] - y_pos[:-1])

    mesh = plsc.VectorSubcoreMesh(core_axis_name="c", subcore_axis_name="s")
    f32 = jnp.float32
    run = pl.kernel(
        _sc_body,
        mesh=mesh,
        compiler_params=pltpu.CompilerParams(needs_layout_passes=False),
        out_type=(jax.ShapeDtypeStruct((N,), f32),
                  jax.ShapeDtypeStruct((N,), f32)),
        scratch_types=[
            pltpu.VMEM((TPAD,), f32),
            pltpu.VMEM((TPAD,), f32),
            pltpu.VMEM((TPAD,), f32),
            pltpu.VMEM((TPAD,), f32),
            pltpu.VMEM((TPAD,), f32),
            pltpu.VMEM((CHUNK,), f32),
            pltpu.VMEM((CHUNK,), f32),
            pltpu.VMEM((CHUNK,), f32),
            pltpu.VMEM((CHUNK,), f32),
            pltpu.VMEM((CHUNK,), f32),
            pltpu.VMEM((CHUNK,), f32),
            pltpu.SemaphoreType.DMA,
            pltpu.SemaphoreType.DMA,
            pltpu.SemaphoreType.DMA,
            pltpu.SemaphoreType.DMA,
            pltpu.SemaphoreType.DMA,
            pltpu.SemaphoreType.DMA,
        ],
    )
    return run(x, xpos_p, ypos_p, d_p, invw_p, h_p)


# speedup vs baseline: 2603.9500x; 1.7861x over previous
import functools

import jax
import jax.numpy as jnp
import numpy as np
from jax import lax
from jax.experimental import pallas as pl
from jax.experimental.pallas import tpu as pltpu
from jax.experimental.pallas import tpu_sc as plsc

RANGE_MIN = -5.0
RANGE_MAX = 5.0
MIN_BIN_SIZE = 0.0001
MIN_SLOPE = 0.0001

LN2 = 0.6931471805599453
SQRT2 = 1.4142135623730951

N = 8388608
NC, NS, L = 2, 16, 16
NW = NC * NS
PER_W = N // NW
CHUNK = 16384
N_CHUNKS = PER_W // CHUNK
N_STEPS = N_CHUNKS // 2
VREGS = CHUNK // L
TPAD = 144


def _log_approx(t):
    bits = lax.bitcast_convert_type(t, jnp.int32)
    e_i = (bits >> 23) - 127
    m = lax.bitcast_convert_type((bits & 0x007FFFFF) | 0x3F800000, jnp.float32)
    big = m >= SQRT2
    m = jnp.where(big, m * 0.5, m)
    e_f = e_i.astype(jnp.float32) + jnp.where(big, 1.0, 0.0)
    z = (m - 1.0) / (m + 1.0)
    z2 = z * z
    p = z * (2.0 + z2 * (2.0 / 3.0 + z2 * (2.0 / 5.0 + z2 * (2.0 / 7.0))))
    return e_f * LN2 + p


def _sc_body(x_hbm, xpos_hbm, ypos_hbm, d_hbm, invw_hbm, h_hbm,
             xposc_hbm, c0_hbm, ln_hbm, rcp_hbm,
             y_hbm, ld_hbm,
             xpos_v, ypos_v, d_v, invw_v, h_v, xposc_v, c0_v, ln_v, rcp_v,
             x0, x1, y0, y1, l0, l1,
             sem_in0, sem_in1, sem_oy0, sem_oy1, sem_ol0, sem_ol1):
    wid = lax.axis_index("s") * NC + lax.axis_index("c")
    base = wid * PER_W

    pltpu.sync_copy(xpos_hbm, xpos_v)
    pltpu.sync_copy(ypos_hbm, ypos_v)
    pltpu.sync_copy(d_hbm, d_v)
    pltpu.sync_copy(invw_hbm, invw_v)
    pltpu.sync_copy(h_hbm, h_v)
    pltpu.sync_copy(xposc_hbm, xposc_v)
    pltpu.sync_copy(c0_hbm, c0_v)
    pltpu.sync_copy(ln_hbm, ln_v)
    pltpu.sync_copy(rcp_hbm, rcp_v)

    coarse = xposc_v[pl.ds(0, L)]

    def make_vreg_body(x_v, y_v, ld_v):
      def vreg_body(off):
        xv = x_v[pl.ds(off, L)]
        c = jnp.zeros((L,), jnp.int32)
        for step in (8, 4, 2, 1):
            cand = c + step
            knot = jnp.take_along_axis(coarse, cand, axis=0)
            c = jnp.where(knot <= xv, cand, c)
        b = c * 8
        for step in (4, 2, 1):
            cand = b + step
            knot = plsc.load_gather(xpos_v, [cand])
            b = jnp.where(knot <= xv, cand, b)
        x_k = plsc.load_gather(xpos_v, [b])
        y_k = plsc.load_gather(ypos_v, [b])
        iw = plsc.load_gather(invw_v, [b])
        hh = plsc.load_gather(h_v, [b])
        d_k = plsc.load_gather(d_v, [b])
        d_k1 = plsc.load_gather(d_v, [b + 1])
        c0 = plsc.load_gather(c0_v, [b])
        s_ = hh * iw
        u = xv - x_k
        xi = jnp.clip(u * iw, 0.0, 1.0)
        om = 1.0 - xi
        xiom = xi * om
        dkom = d_k * om
        num = xi * (s_ * xi + dkom)
        den = s_ + c0 * xiom
        rden = 1.0 / den
        y_spline = y_k + hh * (num * rden)
        numd = s_ * s_ * (d_k1 * xi * xi + (s_ + s_) * xiom + dkom * om)
        deriv = numd * (rden * rden)
        below = xv < RANGE_MIN
        above = xv > RANGE_MAX
        yv = jnp.where(below, (xv - RANGE_MIN) * d_k + RANGE_MIN,
                       jnp.where(above, (xv - RANGE_MAX) * d_k1 + RANGE_MAX,
                                 y_spline))
        bits = lax.bitcast_convert_type(deriv, jnp.int32)
        e_f = ((bits >> 23) - 127).astype(jnp.float32)
        j = (bits >> 16) & 0x7F
        m = lax.bitcast_convert_type((bits & 0x007FFFFF) | 0x3F800000,
                                     jnp.float32)
        delta = m - 1.0 - j.astype(jnp.float32) * (1.0 / 128.0)
        lnm = plsc.load_gather(ln_v, [j]) + delta * plsc.load_gather(rcp_v, [j])
        y_v[pl.ds(off, L)] = yv
        ld_v[pl.ds(off, L)] = e_f * LN2 + lnm
      return vreg_body

    def half(i, g, x_v, y_v, ld_v, sem_in, sem_oy, sem_ol):
        lo = base + g * CHUNK
        out_y = pltpu.make_async_copy(y_v, y_hbm.at[pl.ds(lo, CHUNK)], sem_oy)
        out_l = pltpu.make_async_copy(ld_v, ld_hbm.at[pl.ds(lo, CHUNK)], sem_ol)

        @pl.when(i > 0)
        def _():
            out_y.wait()
            out_l.wait()

        pltpu.make_async_copy(x_hbm.at[pl.ds(lo, CHUNK)], x_v, sem_in).wait()
        plsc.parallel_loop(0, CHUNK, L, unroll=8)(make_vreg_body(x_v, y_v, ld_v))
        out_y.start()
        out_l.start()

        @pl.when(i < N_STEPS - 1)
        def _():
            nxt = lo + 2 * CHUNK
            pltpu.make_async_copy(x_hbm.at[pl.ds(nxt, CHUNK)], x_v, sem_in).start()

    pltpu.make_async_copy(x_hbm.at[pl.ds(base, CHUNK)], x0, sem_in0).start()
    pltpu.make_async_copy(x_hbm.at[pl.ds(base + CHUNK, CHUNK)], x1, sem_in1).start()

    def step(i, carry):
        half(i, 2 * i, x0, y0, l0, sem_in0, sem_oy0, sem_ol0)
        half(i, 2 * i + 1, x1, y1, l1, sem_in1, sem_oy1, sem_ol1)
        return carry

    lax.fori_loop(0, N_STEPS, step, 0)

    tail = base + (N_CHUNKS - 2) * CHUNK
    pltpu.make_async_copy(y0, y_hbm.at[pl.ds(tail, CHUNK)], sem_oy0).wait()
    pltpu.make_async_copy(l0, ld_hbm.at[pl.ds(tail, CHUNK)], sem_ol0).wait()
    pltpu.make_async_copy(y1, y_hbm.at[pl.ds(tail + CHUNK, CHUNK)], sem_oy1).wait()
    pltpu.make_async_copy(l1, ld_hbm.at[pl.ds(tail + CHUNK, CHUNK)], sem_ol1).wait()


@jax.jit
def kernel(x, params):
    K = (params.shape[-1] - 1) // 3
    total_size = RANGE_MAX - RANGE_MIN
    widths = jax.nn.softmax(params[:K]) * (total_size - K * MIN_BIN_SIZE) + MIN_BIN_SIZE
    heights = jax.nn.softmax(params[K:2 * K]) * (total_size - K * MIN_BIN_SIZE) + MIN_BIN_SIZE
    slopes_offset = jnp.log(jnp.exp(1.0 - MIN_SLOPE) - 1.0)
    slopes = jax.nn.softplus(params[2 * K:] + slopes_offset) + MIN_SLOPE
    x_pos = jnp.concatenate([jnp.array([0.0]), jnp.cumsum(widths)]) + RANGE_MIN
    y_pos = jnp.concatenate([jnp.array([0.0]), jnp.cumsum(heights)]) + RANGE_MIN

    def padto(a):
        return jnp.pad(a, (0, TPAD - a.shape[0]), constant_values=1.0).astype(jnp.float32)

    invw = 1.0 / (x_pos[1:] - x_pos[:-1])
    h = y_pos[1:] - y_pos[:-1]
    s_tab = h * invw
    xpos_p = padto(x_pos)
    ypos_p = padto(y_pos)
    d_p = padto(slopes)
    invw_p = padto(invw)
    h_p = padto(h)
    xposc_p = x_pos[0:128:8].astype(jnp.float32)
    c0_p = padto(slopes[1:] + slopes[:-1] - 2.0 * s_tab)
    ln_p = jnp.asarray(np.log1p(np.arange(128) / 128.0), dtype=jnp.float32)
    rcp_p = jnp.asarray(1.0 / (1.0 + np.arange(128) / 128.0), dtype=jnp.float32)

    mesh = plsc.VectorSubcoreMesh(core_axis_name="c", subcore_axis_name="s")
    f32 = jnp.float32
    run = pl.kernel(
        _sc_body,
        mesh=mesh,
        compiler_params=pltpu.CompilerParams(needs_layout_passes=False),
        out_type=(jax.ShapeDtypeStruct((N,), f32),
                  jax.ShapeDtypeStruct((N,), f32)),
        scratch_types=[
            pltpu.VMEM((TPAD,), f32),
            pltpu.VMEM((TPAD,), f32),
            pltpu.VMEM((TPAD,), f32),
            pltpu.VMEM((TPAD,), f32),
            pltpu.VMEM((TPAD,), f32),
            pltpu.VMEM((16,), f32),
            pltpu.VMEM((TPAD,), f32),
            pltpu.VMEM((128,), f32),
            pltpu.VMEM((128,), f32),
            pltpu.VMEM((CHUNK,), f32),
            pltpu.VMEM((CHUNK,), f32),
            pltpu.VMEM((CHUNK,), f32),
            pltpu.VMEM((CHUNK,), f32),
            pltpu.VMEM((CHUNK,), f32),
            pltpu.VMEM((CHUNK,), f32),
            pltpu.SemaphoreType.DMA,
            pltpu.SemaphoreType.DMA,
            pltpu.SemaphoreType.DMA,
            pltpu.SemaphoreType.DMA,
            pltpu.SemaphoreType.DMA,
            pltpu.SemaphoreType.DMA,
        ],
    )
    return run(x, xpos_p, ypos_p, d_p, invw_p, h_p, xposc_p, c0_p, ln_p, rcp_p)


# speedup vs baseline: 2725.6291x; 1.0467x over previous
import functools

import jax
import jax.numpy as jnp
import numpy as np
from jax import lax
from jax.experimental import pallas as pl
from jax.experimental.pallas import tpu as pltpu
from jax.experimental.pallas import tpu_sc as plsc

RANGE_MIN = -5.0
RANGE_MAX = 5.0
MIN_BIN_SIZE = 0.0001
MIN_SLOPE = 0.0001

LN2 = 0.6931471805599453
SQRT2 = 1.4142135623730951

N = 8388608
NC, NS, L = 2, 16, 16
NW = NC * NS
PER_W = N // NW
CHUNK = 16384
N_CHUNKS = PER_W // CHUNK
N_STEPS = N_CHUNKS // 2
VREGS = CHUNK // L
TPAD = 144


def _log_approx(t):
    bits = lax.bitcast_convert_type(t, jnp.int32)
    e_i = (bits >> 23) - 127
    m = lax.bitcast_convert_type((bits & 0x007FFFFF) | 0x3F800000, jnp.float32)
    big = m >= SQRT2
    m = jnp.where(big, m * 0.5, m)
    e_f = e_i.astype(jnp.float32) + jnp.where(big, 1.0, 0.0)
    z = (m - 1.0) / (m + 1.0)
    z2 = z * z
    p = z * (2.0 + z2 * (2.0 / 3.0 + z2 * (2.0 / 5.0 + z2 * (2.0 / 7.0))))
    return e_f * LN2 + p


def _sc_body(x_hbm, xpos_hbm, ypos_hbm, d_hbm, invw_hbm, h_hbm, t0_hbm,
             xposc_hbm, c0_hbm, ln_hbm, rcp_hbm,
             y_hbm, ld_hbm,
             xpos_v, ypos_v, d_v, invw_v, h_v, t0_v, xposc_v, c0_v, ln_v, rcp_v,
             x0, x1, y0, y1, l0, l1,
             sem_in0, sem_in1, sem_oy0, sem_oy1, sem_ol0, sem_ol1):
    wid = lax.axis_index("s") * NC + lax.axis_index("c")
    base = wid * PER_W

    pltpu.sync_copy(xpos_hbm, xpos_v)
    pltpu.sync_copy(ypos_hbm, ypos_v)
    pltpu.sync_copy(d_hbm, d_v)
    pltpu.sync_copy(invw_hbm, invw_v)
    pltpu.sync_copy(h_hbm, h_v)
    pltpu.sync_copy(t0_hbm, t0_v)
    pltpu.sync_copy(xposc_hbm, xposc_v)
    pltpu.sync_copy(c0_hbm, c0_v)
    pltpu.sync_copy(ln_hbm, ln_v)
    pltpu.sync_copy(rcp_hbm, rcp_v)

    coarse = xposc_v[pl.ds(0, L)]

    def make_vreg_body(x_v, y_v, ld_v):
      def vreg_body(off):
        xv = x_v[pl.ds(off, L)]
        c = jnp.zeros((L,), jnp.int32)
        for step in (8, 4, 2, 1):
            cand = c + step
            knot = jnp.take_along_axis(coarse, cand, axis=0)
            c = jnp.where(knot <= xv, cand, c)
        b = c * 8
        for step in (4, 2, 1):
            cand = b + step
            knot = plsc.load_gather(xpos_v, [cand])
            b = jnp.where(knot <= xv, cand, b)
        t0 = plsc.load_gather(t0_v, [b])
        y_k = plsc.load_gather(ypos_v, [b])
        iw = plsc.load_gather(invw_v, [b])
        hh = plsc.load_gather(h_v, [b])
        d_k = plsc.load_gather(d_v, [b])
        d_k1 = plsc.load_gather(d_v, [b + 1])
        c0 = plsc.load_gather(c0_v, [b])
        s_ = hh * iw
        xi = jnp.clip(xv * iw + t0, 0.0, 1.0)
        om = 1.0 - xi
        xiom = xi * om
        dkom = d_k * om
        num = xi * (s_ * xi + dkom)
        den = s_ + c0 * xiom
        rden = 1.0 / den
        y_spline = y_k + hh * (num * rden)
        numd = s_ * s_ * (d_k1 * xi * xi + (s_ + s_) * xiom + dkom * om)
        deriv = numd * (rden * rden)
        below = xv < RANGE_MIN
        above = xv > RANGE_MAX
        yv = jnp.where(below, (xv - RANGE_MIN) * d_k + RANGE_MIN,
                       jnp.where(above, (xv - RANGE_MAX) * d_k1 + RANGE_MAX,
                                 y_spline))
        bits = lax.bitcast_convert_type(deriv, jnp.int32)
        e_f = (bits >> 23).astype(jnp.float32)
        j = (bits >> 16) & 0x7F
        f_cvt = (bits & 0xFFFF).astype(jnp.float32)
        lnm = plsc.load_gather(ln_v, [j]) + f_cvt * plsc.load_gather(rcp_v, [j])
        y_v[pl.ds(off, L)] = yv
        ld_v[pl.ds(off, L)] = e_f * LN2 + lnm
      return vreg_body

    def half(i, g, x_v, y_v, ld_v, sem_in, sem_oy, sem_ol):
        lo = base + g * CHUNK
        out_y = pltpu.make_async_copy(y_v, y_hbm.at[pl.ds(lo, CHUNK)], sem_oy)
        out_l = pltpu.make_async_copy(ld_v, ld_hbm.at[pl.ds(lo, CHUNK)], sem_ol)

        @pl.when(i > 0)
        def _():
            out_y.wait()
            out_l.wait()

        pltpu.make_async_copy(x_hbm.at[pl.ds(lo, CHUNK)], x_v, sem_in).wait()
        plsc.parallel_loop(0, CHUNK, L, unroll=8)(make_vreg_body(x_v, y_v, ld_v))
        out_y.start()
        out_l.start()

        @pl.when(i < N_STEPS - 1)
        def _():
            nxt = lo + 2 * CHUNK
            pltpu.make_async_copy(x_hbm.at[pl.ds(nxt, CHUNK)], x_v, sem_in).start()

    pltpu.make_async_copy(x_hbm.at[pl.ds(base, CHUNK)], x0, sem_in0).start()
    pltpu.make_async_copy(x_hbm.at[pl.ds(base + CHUNK, CHUNK)], x1, sem_in1).start()

    def step(i, carry):
        half(i, 2 * i, x0, y0, l0, sem_in0, sem_oy0, sem_ol0)
        half(i, 2 * i + 1, x1, y1, l1, sem_in1, sem_oy1, sem_ol1)
        return carry

    lax.fori_loop(0, N_STEPS, step, 0)

    tail = base + (N_CHUNKS - 2) * CHUNK
    pltpu.make_async_copy(y0, y_hbm.at[pl.ds(tail, CHUNK)], sem_oy0).wait()
    pltpu.make_async_copy(l0, ld_hbm.at[pl.ds(tail, CHUNK)], sem_ol0).wait()
    pltpu.make_async_copy(y1, y_hbm.at[pl.ds(tail + CHUNK, CHUNK)], sem_oy1).wait()
    pltpu.make_async_copy(l1, ld_hbm.at[pl.ds(tail + CHUNK, CHUNK)], sem_ol1).wait()


@jax.jit
def kernel(x, params):
    K = (params.shape[-1] - 1) // 3
    total_size = RANGE_MAX - RANGE_MIN
    widths = jax.nn.softmax(params[:K]) * (total_size - K * MIN_BIN_SIZE) + MIN_BIN_SIZE
    heights = jax.nn.softmax(params[K:2 * K]) * (total_size - K * MIN_BIN_SIZE) + MIN_BIN_SIZE
    slopes_offset = jnp.log(jnp.exp(1.0 - MIN_SLOPE) - 1.0)
    slopes = jax.nn.softplus(params[2 * K:] + slopes_offset) + MIN_SLOPE
    x_pos = jnp.concatenate([jnp.array([0.0]), jnp.cumsum(widths)]) + RANGE_MIN
    y_pos = jnp.concatenate([jnp.array([0.0]), jnp.cumsum(heights)]) + RANGE_MIN

    def padto(a):
        return jnp.pad(a, (0, TPAD - a.shape[0]), constant_values=1.0).astype(jnp.float32)

    invw = 1.0 / (x_pos[1:] - x_pos[:-1])
    h = y_pos[1:] - y_pos[:-1]
    s_tab = h * invw
    xpos_p = padto(x_pos)
    ypos_p = padto(y_pos)
    d_p = padto(slopes)
    invw_p = padto(invw)
    h_p = padto(h)
    t0_p = padto(-x_pos[:128] * invw)
    xposc_p = x_pos[0:128:8].astype(jnp.float32)
    c0_p = padto(slopes[1:] + slopes[:-1] - 2.0 * s_tab)
    ln_p = jnp.asarray(np.log1p(np.arange(128) / 128.0) - 127.0 * np.log(2.0),
                       dtype=jnp.float32)
    rcp_p = jnp.asarray(2.0 ** -23 / (1.0 + np.arange(128) / 128.0),
                        dtype=jnp.float32)

    mesh = plsc.VectorSubcoreMesh(core_axis_name="c", subcore_axis_name="s")
    f32 = jnp.float32
    run = pl.kernel(
        _sc_body,
        mesh=mesh,
        compiler_params=pltpu.CompilerParams(needs_layout_passes=False),
        out_type=(jax.ShapeDtypeStruct((N,), f32),
                  jax.ShapeDtypeStruct((N,), f32)),
        scratch_types=[
            pltpu.VMEM((TPAD,), f32),
            pltpu.VMEM((TPAD,), f32),
            pltpu.VMEM((TPAD,), f32),
            pltpu.VMEM((TPAD,), f32),
            pltpu.VMEM((TPAD,), f32),
            pltpu.VMEM((TPAD,), f32),
            pltpu.VMEM((16,), f32),
            pltpu.VMEM((TPAD,), f32),
            pltpu.VMEM((128,), f32),
            pltpu.VMEM((128,), f32),
            pltpu.VMEM((CHUNK,), f32),
            pltpu.VMEM((CHUNK,), f32),
            pltpu.VMEM((CHUNK,), f32),
            pltpu.VMEM((CHUNK,), f32),
            pltpu.VMEM((CHUNK,), f32),
            pltpu.VMEM((CHUNK,), f32),
            pltpu.SemaphoreType.DMA,
            pltpu.SemaphoreType.DMA,
            pltpu.SemaphoreType.DMA,
            pltpu.SemaphoreType.DMA,
            pltpu.SemaphoreType.DMA,
            pltpu.SemaphoreType.DMA,
        ],
    )
    return run(x, xpos_p, ypos_p, d_p, invw_p, h_p, t0_p,
               xposc_p, c0_p, ln_p, rcp_p)


# speedup vs baseline: 2872.0151x; 1.0537x over previous
import functools

import jax
import jax.numpy as jnp
import numpy as np
from jax import lax
from jax.experimental import pallas as pl
from jax.experimental.pallas import tpu as pltpu
from jax.experimental.pallas import tpu_sc as plsc

RANGE_MIN = -5.0
RANGE_MAX = 5.0
MIN_BIN_SIZE = 0.0001
MIN_SLOPE = 0.0001

LN2 = 0.6931471805599453
SQRT2 = 1.4142135623730951

N = 8388608
NC, NS, L = 2, 16, 16
NW = NC * NS
PER_W = N // NW
CHUNK = 16384
N_CHUNKS = PER_W // CHUNK
N_STEPS = N_CHUNKS // 2
VREGS = CHUNK // L
TPAD = 144


def _log_approx(t):
    bits = lax.bitcast_convert_type(t, jnp.int32)
    e_i = (bits >> 23) - 127
    m = lax.bitcast_convert_type((bits & 0x007FFFFF) | 0x3F800000, jnp.float32)
    big = m >= SQRT2
    m = jnp.where(big, m * 0.5, m)
    e_f = e_i.astype(jnp.float32) + jnp.where(big, 1.0, 0.0)
    z = (m - 1.0) / (m + 1.0)
    z2 = z * z
    p = z * (2.0 + z2 * (2.0 / 3.0 + z2 * (2.0 / 5.0 + z2 * (2.0 / 7.0))))
    return e_f * LN2 + p


def _sc_body(x_hbm, xpos_hbm, ypos_hbm, d_hbm, invw_hbm, h_hbm, t0_hbm,
             xposc_hbm, c0_hbm, ln_hbm, rcp_hbm,
             y_hbm, ld_hbm,
             xpos_v, ypos_v, d_v, invw_v, h_v, t0_v, xposc_v, c0_v, ln_v, rcp_v,
             x0, x1, y0, y1, l0, l1,
             sem_in0, sem_in1, sem_oy0, sem_oy1, sem_ol0, sem_ol1):
    wid = lax.axis_index("s") * NC + lax.axis_index("c")
    base = wid * PER_W

    pltpu.sync_copy(xpos_hbm, xpos_v)
    pltpu.sync_copy(ypos_hbm, ypos_v)
    pltpu.sync_copy(d_hbm, d_v)
    pltpu.sync_copy(invw_hbm, invw_v)
    pltpu.sync_copy(h_hbm, h_v)
    pltpu.sync_copy(t0_hbm, t0_v)
    pltpu.sync_copy(xposc_hbm, xposc_v)
    pltpu.sync_copy(c0_hbm, c0_v)
    pltpu.sync_copy(ln_hbm, ln_v)
    pltpu.sync_copy(rcp_hbm, rcp_v)

    coarse = xposc_v[pl.ds(0, L)]

    def make_vreg_body(x_v, y_v, ld_v):
      def vreg_body(off):
        xv = x_v[pl.ds(off, L)]
        c = jnp.zeros((L,), jnp.int32)
        for step in (8, 4, 2, 1):
            cand = c + step
            knot = jnp.take_along_axis(coarse, cand, axis=0)
            c = jnp.where(knot <= xv, cand, c)
        b = c * 8
        for step in (4, 2, 1):
            cand = b + step
            knot = plsc.load_gather(xpos_v, [cand])
            b = jnp.where(knot <= xv, cand, b)
        t0 = plsc.load_gather(t0_v, [b])
        y_k = plsc.load_gather(ypos_v, [b])
        iw = plsc.load_gather(invw_v, [b])
        hh = plsc.load_gather(h_v, [b])
        d_k = plsc.load_gather(d_v, [b])
        d_k1 = plsc.load_gather(d_v, [b + 1])
        c0 = plsc.load_gather(c0_v, [b])
        s_ = hh * iw
        xi = jnp.clip(xv * iw + t0, 0.0, 1.0)
        om = 1.0 - xi
        xiom = xi * om
        dkom = d_k * om
        num = xi * (s_ * xi + dkom)
        den = s_ + c0 * xiom
        rden = 1.0 / den
        y_spline = y_k + hh * (num * rden)
        numd = s_ * s_ * (d_k1 * xi * xi + (s_ + s_) * xiom + dkom * om)
        deriv = numd * (rden * rden)
        below = xv < RANGE_MIN
        above = xv > RANGE_MAX
        yv = jnp.where(below, (xv - RANGE_MIN) * d_k + RANGE_MIN,
                       jnp.where(above, (xv - RANGE_MAX) * d_k1 + RANGE_MAX,
                                 y_spline))
        bits = lax.bitcast_convert_type(deriv, jnp.int32)
        e_f = (bits >> 23).astype(jnp.float32)
        j = (bits >> 16) & 0x7F
        f_cvt = (bits & 0xFFFF).astype(jnp.float32)
        lnm = plsc.load_gather(ln_v, [j]) + f_cvt * plsc.load_gather(rcp_v, [j])
        y_v[pl.ds(off, L)] = yv
        ld_v[pl.ds(off, L)] = e_f * LN2 + lnm
      return vreg_body

    def half(i, g, x_v, y_v, ld_v, sem_in, sem_oy, sem_ol):
        lo = base + g * CHUNK
        out_y = pltpu.make_async_copy(y_v, y_hbm.at[pl.ds(lo, CHUNK)], sem_oy)
        out_l = pltpu.make_async_copy(ld_v, ld_hbm.at[pl.ds(lo, CHUNK)], sem_ol)

        @pl.when(i > 0)
        def _():
            out_y.wait()
            out_l.wait()

        pltpu.make_async_copy(x_hbm.at[pl.ds(lo, CHUNK)], x_v, sem_in).wait()
        plsc.parallel_loop(0, CHUNK, L, unroll=12)(make_vreg_body(x_v, y_v, ld_v))
        out_y.start()
        out_l.start()

        @pl.when(i < N_STEPS - 1)
        def _():
            nxt = lo + 2 * CHUNK
            pltpu.make_async_copy(x_hbm.at[pl.ds(nxt, CHUNK)], x_v, sem_in).start()

    pltpu.make_async_copy(x_hbm.at[pl.ds(base, CHUNK)], x0, sem_in0).start()
    pltpu.make_async_copy(x_hbm.at[pl.ds(base + CHUNK, CHUNK)], x1, sem_in1).start()

    def step(i, carry):
        half(i, 2 * i, x0, y0, l0, sem_in0, sem_oy0, sem_ol0)
        half(i, 2 * i + 1, x1, y1, l1, sem_in1, sem_oy1, sem_ol1)
        return carry

    lax.fori_loop(0, N_STEPS, step, 0)

    tail = base + (N_CHUNKS - 2) * CHUNK
    pltpu.make_async_copy(y0, y_hbm.at[pl.ds(tail, CHUNK)], sem_oy0).wait()
    pltpu.make_async_copy(l0, ld_hbm.at[pl.ds(tail, CHUNK)], sem_ol0).wait()
    pltpu.make_async_copy(y1, y_hbm.at[pl.ds(tail + CHUNK, CHUNK)], sem_oy1).wait()
    pltpu.make_async_copy(l1, ld_hbm.at[pl.ds(tail + CHUNK, CHUNK)], sem_ol1).wait()


@jax.jit
def kernel(x, params):
    K = (params.shape[-1] - 1) // 3
    total_size = RANGE_MAX - RANGE_MIN
    widths = jax.nn.softmax(params[:K]) * (total_size - K * MIN_BIN_SIZE) + MIN_BIN_SIZE
    heights = jax.nn.softmax(params[K:2 * K]) * (total_size - K * MIN_BIN_SIZE) + MIN_BIN_SIZE
    slopes_offset = jnp.log(jnp.exp(1.0 - MIN_SLOPE) - 1.0)
    slopes = jax.nn.softplus(params[2 * K:] + slopes_offset) + MIN_SLOPE
    x_pos = jnp.concatenate([jnp.array([0.0]), jnp.cumsum(widths)]) + RANGE_MIN
    y_pos = jnp.concatenate([jnp.array([0.0]), jnp.cumsum(heights)]) + RANGE_MIN

    def padto(a):
        return jnp.pad(a, (0, TPAD - a.shape[0]), constant_values=1.0).astype(jnp.float32)

    invw = 1.0 / (x_pos[1:] - x_pos[:-1])
    h = y_pos[1:] - y_pos[:-1]
    s_tab = h * invw
    xpos_p = padto(x_pos)
    ypos_p = padto(y_pos)
    d_p = padto(slopes)
    invw_p = padto(invw)
    h_p = padto(h)
    t0_p = padto(-x_pos[:128] * invw)
    xposc_p = x_pos[0:128:8].astype(jnp.float32)
    c0_p = padto(slopes[1:] + slopes[:-1] - 2.0 * s_tab)
    ln_p = jnp.asarray(np.log1p(np.arange(128) / 128.0) - 127.0 * np.log(2.0),
                       dtype=jnp.float32)
    rcp_p = jnp.asarray(2.0 ** -23 / (1.0 + np.arange(128) / 128.0),
                        dtype=jnp.float32)

    mesh = plsc.VectorSubcoreMesh(core_axis_name="c", subcore_axis_name="s")
    f32 = jnp.float32
    run = pl.kernel(
        _sc_body,
        mesh=mesh,
        compiler_params=pltpu.CompilerParams(needs_layout_passes=False),
        out_type=(jax.ShapeDtypeStruct((N,), f32),
                  jax.ShapeDtypeStruct((N,), f32)),
        scratch_types=[
            pltpu.VMEM((TPAD,), f32),
            pltpu.VMEM((TPAD,), f32),
            pltpu.VMEM((TPAD,), f32),
            pltpu.VMEM((TPAD,), f32),
            pltpu.VMEM((TPAD,), f32),
            pltpu.VMEM((TPAD,), f32),
            pltpu.VMEM((16,), f32),
            pltpu.VMEM((TPAD,), f32),
            pltpu.VMEM((128,), f32),
            pltpu.VMEM((128,), f32),
            pltpu.VMEM((CHUNK,), f32),
            pltpu.VMEM((CHUNK,), f32),
            pltpu.VMEM((CHUNK,), f32),
            pltpu.VMEM((CHUNK,), f32),
            pltpu.VMEM((CHUNK,), f32),
            pltpu.VMEM((CHUNK,), f32),
            pltpu.SemaphoreType.DMA,
            pltpu.SemaphoreType.DMA,
            pltpu.SemaphoreType.DMA,
            pltpu.SemaphoreType.DMA,
            pltpu.SemaphoreType.DMA,
            pltpu.SemaphoreType.DMA,
        ],
    )
    return run(x, xpos_p, ypos_p, d_p, invw_p, h_p, t0_p,
               xposc_p, c0_p, ln_p, rcp_p)


# speedup vs baseline: 3001.4266x; 1.0451x over previous
import functools

import jax
import jax.numpy as jnp
import numpy as np
from jax import lax
from jax.experimental import pallas as pl
from jax.experimental.pallas import tpu as pltpu
from jax.experimental.pallas import tpu_sc as plsc

RANGE_MIN = -5.0
RANGE_MAX = 5.0
MIN_BIN_SIZE = 0.0001
MIN_SLOPE = 0.0001

LN2 = 0.6931471805599453
SQRT2 = 1.4142135623730951

N = 8388608
NC, NS, L = 2, 16, 16
NW = NC * NS
PER_W = N // NW
CHUNK = 16384
N_CHUNKS = PER_W // CHUNK
N_STEPS = N_CHUNKS // 2
VREGS = CHUNK // L
TPAD = 144


def _log_approx(t):
    bits = lax.bitcast_convert_type(t, jnp.int32)
    e_i = (bits >> 23) - 127
    m = lax.bitcast_convert_type((bits & 0x007FFFFF) | 0x3F800000, jnp.float32)
    big = m >= SQRT2
    m = jnp.where(big, m * 0.5, m)
    e_f = e_i.astype(jnp.float32) + jnp.where(big, 1.0, 0.0)
    z = (m - 1.0) / (m + 1.0)
    z2 = z * z
    p = z * (2.0 + z2 * (2.0 / 3.0 + z2 * (2.0 / 5.0 + z2 * (2.0 / 7.0))))
    return e_f * LN2 + p


def _sc_body(x_hbm, xpos_hbm, ypos_hbm, d_hbm, invw_hbm, h_hbm, t0_hbm,
             xposc_hbm, c0_hbm, ln_hbm, rcp_hbm,
             y_hbm, ld_hbm,
             xpos_v, ypos_v, d_v, invw_v, h_v, t0_v, xposc_v, c0_v, ln_v, rcp_v,
             x0, x1, y0, y1, l0, l1,
             sem_in0, sem_in1, sem_oy0, sem_oy1, sem_ol0, sem_ol1):
    wid = lax.axis_index("s") * NC + lax.axis_index("c")
    base = wid * PER_W

    pltpu.sync_copy(xpos_hbm, xpos_v)
    pltpu.sync_copy(ypos_hbm, ypos_v)
    pltpu.sync_copy(d_hbm, d_v)
    pltpu.sync_copy(invw_hbm, invw_v)
    pltpu.sync_copy(h_hbm, h_v)
    pltpu.sync_copy(t0_hbm, t0_v)
    pltpu.sync_copy(xposc_hbm, xposc_v)
    pltpu.sync_copy(c0_hbm, c0_v)
    pltpu.sync_copy(ln_hbm, ln_v)
    pltpu.sync_copy(rcp_hbm, rcp_v)

    coarse = xposc_v[pl.ds(0, L)]

    def make_vreg_body(x_v, y_v, ld_v):
      def vreg_body(off):
        xv = x_v[pl.ds(off, L)]
        c = jnp.zeros((L,), jnp.int32)
        for step in (8, 4, 2, 1):
            cand = c + step
            knot = jnp.take_along_axis(coarse, cand, axis=0)
            c = jnp.where(knot <= xv, cand, c)
        b = c * 8
        for step in (4, 2, 1):
            cand = b + step
            knot = plsc.load_gather(xpos_v, [cand])
            b = jnp.where(knot <= xv, cand, b)
        t0 = plsc.load_gather(t0_v, [b])
        y_k = plsc.load_gather(ypos_v, [b])
        iw = plsc.load_gather(invw_v, [b])
        hh = plsc.load_gather(h_v, [b])
        d_k = plsc.load_gather(d_v, [b])
        d_k1 = plsc.load_gather(d_v, [b + 1])
        c0 = plsc.load_gather(c0_v, [b])
        s_ = hh * iw
        xi = jnp.clip(xv * iw + t0, 0.0, 1.0)
        om = 1.0 - xi
        xiom = xi * om
        dkom = d_k * om
        num = xi * (s_ * xi + dkom)
        den = s_ + c0 * xiom
        rden = 1.0 / den
        y_spline = y_k + hh * (num * rden)
        numd = s_ * s_ * (d_k1 * xi * xi + (s_ + s_) * xiom + dkom * om)
        deriv = numd * (rden * rden)
        below = xv < RANGE_MIN
        above = xv > RANGE_MAX
        yv = jnp.where(below, (xv - RANGE_MIN) * d_k + RANGE_MIN,
                       jnp.where(above, (xv - RANGE_MAX) * d_k1 + RANGE_MAX,
                                 y_spline))
        bits = lax.bitcast_convert_type(deriv, jnp.int32)
        e_f = (bits >> 23).astype(jnp.float32)
        j = (bits >> 16) & 0x7F
        f_cvt = (bits & 0xFFFF).astype(jnp.float32)
        lnm = plsc.load_gather(ln_v, [j]) + f_cvt * plsc.load_gather(rcp_v, [j])
        y_v[pl.ds(off, L)] = yv
        ld_v[pl.ds(off, L)] = e_f * LN2 + lnm
      return vreg_body

    def half(i, g, x_v, y_v, ld_v, sem_in, sem_oy, sem_ol):
        lo = base + g * CHUNK
        out_y = pltpu.make_async_copy(y_v, y_hbm.at[pl.ds(lo, CHUNK)], sem_oy)
        out_l = pltpu.make_async_copy(ld_v, ld_hbm.at[pl.ds(lo, CHUNK)], sem_ol)

        @pl.when(i > 0)
        def _():
            out_y.wait()
            out_l.wait()

        pltpu.make_async_copy(x_hbm.at[pl.ds(lo, CHUNK)], x_v, sem_in).wait()
        plsc.parallel_loop(0, CHUNK, L, unroll=16)(make_vreg_body(x_v, y_v, ld_v))
        out_y.start()
        out_l.start()

        @pl.when(i < N_STEPS - 1)
        def _():
            nxt = lo + 2 * CHUNK
            pltpu.make_async_copy(x_hbm.at[pl.ds(nxt, CHUNK)], x_v, sem_in).start()

    pltpu.make_async_copy(x_hbm.at[pl.ds(base, CHUNK)], x0, sem_in0).start()
    pltpu.make_async_copy(x_hbm.at[pl.ds(base + CHUNK, CHUNK)], x1, sem_in1).start()

    def step(i, carry):
        half(i, 2 * i, x0, y0, l0, sem_in0, sem_oy0, sem_ol0)
        half(i, 2 * i + 1, x1, y1, l1, sem_in1, sem_oy1, sem_ol1)
        return carry

    lax.fori_loop(0, N_STEPS, step, 0)

    tail = base + (N_CHUNKS - 2) * CHUNK
    pltpu.make_async_copy(y0, y_hbm.at[pl.ds(tail, CHUNK)], sem_oy0).wait()
    pltpu.make_async_copy(l0, ld_hbm.at[pl.ds(tail, CHUNK)], sem_ol0).wait()
    pltpu.make_async_copy(y1, y_hbm.at[pl.ds(tail + CHUNK, CHUNK)], sem_oy1).wait()
    pltpu.make_async_copy(l1, ld_hbm.at[pl.ds(tail + CHUNK, CHUNK)], sem_ol1).wait()


@jax.jit
def kernel(x, params):
    K = (params.shape[-1] - 1) // 3
    total_size = RANGE_MAX - RANGE_MIN
    widths = jax.nn.softmax(params[:K]) * (total_size - K * MIN_BIN_SIZE) + MIN_BIN_SIZE
    heights = jax.nn.softmax(params[K:2 * K]) * (total_size - K * MIN_BIN_SIZE) + MIN_BIN_SIZE
    slopes_offset = jnp.log(jnp.exp(1.0 - MIN_SLOPE) - 1.0)
    slopes = jax.nn.softplus(params[2 * K:] + slopes_offset) + MIN_SLOPE
    x_pos = jnp.concatenate([jnp.array([0.0]), jnp.cumsum(widths)]) + RANGE_MIN
    y_pos = jnp.concatenate([jnp.array([0.0]), jnp.cumsum(heights)]) + RANGE_MIN

    def padto(a):
        return jnp.pad(a, (0, TPAD - a.shape[0]), constant_values=1.0).astype(jnp.float32)

    invw = 1.0 / (x_pos[1:] - x_pos[:-1])
    h = y_pos[1:] - y_pos[:-1]
    s_tab = h * invw
    xpos_p = padto(x_pos)
    ypos_p = padto(y_pos)
    d_p = padto(slopes)
    invw_p = padto(invw)
    h_p = padto(h)
    t0_p = padto(-x_pos[:128] * invw)
    xposc_p = x_pos[0:128:8].astype(jnp.float32)
    c0_p = padto(slopes[1:] + slopes[:-1] - 2.0 * s_tab)
    ln_p = jnp.asarray(np.log1p(np.arange(128) / 128.0) - 127.0 * np.log(2.0),
                       dtype=jnp.float32)
    rcp_p = jnp.asarray(2.0 ** -23 / (1.0 + np.arange(128) / 128.0),
                        dtype=jnp.float32)

    mesh = plsc.VectorSubcoreMesh(core_axis_name="c", subcore_axis_name="s")
    f32 = jnp.float32
    run = pl.kernel(
        _sc_body,
        mesh=mesh,
        compiler_params=pltpu.CompilerParams(needs_layout_passes=False),
        out_type=(jax.ShapeDtypeStruct((N,), f32),
                  jax.ShapeDtypeStruct((N,), f32)),
        scratch_types=[
            pltpu.VMEM((TPAD,), f32),
            pltpu.VMEM((TPAD,), f32),
            pltpu.VMEM((TPAD,), f32),
            pltpu.VMEM((TPAD,), f32),
            pltpu.VMEM((TPAD,), f32),
            pltpu.VMEM((TPAD,), f32),
            pltpu.VMEM((16,), f32),
            pltpu.VMEM((TPAD,), f32),
            pltpu.VMEM((128,), f32),
            pltpu.VMEM((128,), f32),
            pltpu.VMEM((CHUNK,), f32),
            pltpu.VMEM((CHUNK,), f32),
            pltpu.VMEM((CHUNK,), f32),
            pltpu.VMEM((CHUNK,), f32),
            pltpu.VMEM((CHUNK,), f32),
            pltpu.VMEM((CHUNK,), f32),
            pltpu.SemaphoreType.DMA,
            pltpu.SemaphoreType.DMA,
            pltpu.SemaphoreType.DMA,
            pltpu.SemaphoreType.DMA,
            pltpu.SemaphoreType.DMA,
            pltpu.SemaphoreType.DMA,
        ],
    )
    return run(x, xpos_p, ypos_p, d_p, invw_p, h_p, t0_p,
               xposc_p, c0_p, ln_p, rcp_p)
